# baseline (speedup 1.0000x reference)
import functools

import jax
import jax.numpy as jnp
from jax import lax
from jax.experimental import pallas as pl
from jax.experimental.pallas import tpu as pltpu
from jax.experimental.pallas import tpu_sc as plsc

ENT_TOTAL = 100000
REL_TOTAL = 100000
HIDDEN = 64
BATCH_SIZE = 16384
BATCH_SEQ_SIZE = 32768
MARGIN = 1.0
EPS = 0.001
PD_EPS = 1e-6

NC, NS, L = 2, 16, 16
NW = NC * NS
PER_W = BATCH_SEQ_SIZE // NW
G = 128
NCHUNK = PER_W // G
D = HIDDEN


def _sc_body(hidx_hbm, ridx_hbm, tidx_hbm, ent_hbm, rel_hbm, norm_hbm, out_hbm,
             hidx_v, ridx_v, tidx_v,
             h0, t0, r0, n0, h1, t1, r1, n1, o_v, sem0, sem1):
    wid = lax.axis_index("s") * NC + lax.axis_index("c")
    base = wid * PER_W
    bufs = [(h0, t0, r0, n0), (h1, t1, r1, n1)]
    sems = [sem0, sem1]

    pltpu.sync_copy(hidx_hbm.at[pl.ds(base, PER_W)], hidx_v)
    pltpu.sync_copy(ridx_hbm.at[pl.ds(base, PER_W)], ridx_v)
    pltpu.sync_copy(tidx_hbm.at[pl.ds(base, PER_W)], tidx_v)

    def issue(g):
        p = g % 2
        hb, tb, rb, nb = bufs[p]
        sl = pl.ds(g * G, G)
        return [
            pltpu.async_copy(ent_hbm.at[hidx_v.at[sl]], hb, sems[p]),
            pltpu.async_copy(ent_hbm.at[tidx_v.at[sl]], tb, sems[p]),
            pltpu.async_copy(rel_hbm.at[ridx_v.at[sl]], rb, sems[p]),
            pltpu.async_copy(norm_hbm.at[ridx_v.at[sl]], nb, sems[p]),
        ]

    lane = lax.iota(jnp.int32, L)
    pending = issue(0)
    for g in range(NCHUNK):
        nxt = issue(g + 1) if g + 1 < NCHUNK else []
        for c in pending:
            c.wait()
        pending = nxt
        hb, tb, rb, nb = bufs[g % 2]

        def group(k, _):
            def triple(q, packed):
                i = k * L + q
                z = jnp.zeros((L,), jnp.float32)
                A, U, N, HT = z, z, z, z
                for c in range(D // L):
                    sl = pl.ds(c * L, L)
                    h = hb[i, sl]
                    t = tb[i, sl]
                    r = rb[i, sl]
                    n = nb[i, sl]
                    d = h - t
                    u = d + r + PD_EPS
                    A = A + u * u
                    U = U + u * n
                    N = N + n * n
                    HT = HT + d * n
                As = jnp.full((L,), jnp.sum(A))
                Us = jnp.full((L,), jnp.sum(U))
                Ns = jnp.full((L,), jnp.sum(N))
                HTs = jnp.full((L,), jnp.sum(HT))
                beta = HTs / jnp.maximum(Ns, 1e-24)
                sc2 = As - 2.0 * beta * Us + beta * beta * Ns
                sc2 = jnp.maximum(sc2, 0.0)
                return jnp.where(lane == q, sc2, packed)

            packed = lax.fori_loop(0, L, triple, jnp.zeros((L,), jnp.float32))
            o_v[pl.ds(g * G + k * L, L)] = packed
            return 0

        lax.fori_loop(0, G // L, group, 0)

    pltpu.sync_copy(o_v, out_hbm.at[pl.ds(base, PER_W)])


@functools.lru_cache(maxsize=1)
def _build_sc_scores():
    return pl.kernel(
        _sc_body,
        mesh=plsc.VectorSubcoreMesh(core_axis_name="c", subcore_axis_name="s"),
        compiler_params=pltpu.CompilerParams(
            needs_layout_passes=False, use_tc_tiling_on_sc=False
        ),
        out_type=jax.ShapeDtypeStruct((BATCH_SEQ_SIZE,), jnp.float32),
        scratch_types=[
            pltpu.VMEM((PER_W,), jnp.int32),
            pltpu.VMEM((PER_W,), jnp.int32),
            pltpu.VMEM((PER_W,), jnp.int32),
            pltpu.VMEM((G, D), jnp.float32),
            pltpu.VMEM((G, D), jnp.float32),
            pltpu.VMEM((G, D), jnp.float32),
            pltpu.VMEM((G, D), jnp.float32),
            pltpu.VMEM((G, D), jnp.float32),
            pltpu.VMEM((G, D), jnp.float32),
            pltpu.VMEM((G, D), jnp.float32),
            pltpu.VMEM((G, D), jnp.float32),
            pltpu.VMEM((PER_W,), jnp.float32),
            pltpu.SemaphoreType.DMA,
            pltpu.SemaphoreType.DMA,
        ],
    )


DCHUNK = 8
GRID_TC = D // DCHUNK


def _tc_body(ent_ref, rel_ref, norm_ref, out_ref, ss_acc, dot_acc, rr_acc):
    i = pl.program_id(0)
    e = ent_ref[...]
    r = rel_ref[...]
    nw = norm_ref[...]

    @pl.when(i == 0)
    def _first():
        ss_acc[...] = e * e
        dot_acc[...] = nw * r
        rr_acc[...] = r * r

    @pl.when(i > 0)
    def _rest():
        ss_acc[...] += e * e
        dot_acc[...] += nw * r
        rr_acc[...] += r * r

    @pl.when(i == GRID_TC - 1)
    def _final():
        ss = jnp.maximum(jnp.sum(ss_acc[...], axis=0), 1.0)
        ent_part = jnp.sum(ss * lax.rsqrt(ss) - 1.0)
        orth = jnp.sum(dot_acc[...], axis=0) * lax.rsqrt(jnp.sum(rr_acc[...], axis=0))
        orth_part = jnp.sum(jnp.maximum(orth - EPS * EPS, 0.0))
        out_ref[...] = jnp.reshape(ent_part / ENT_TOTAL + orth_part / REL_TOTAL, (1, 1))


_tc_losses = pl.pallas_call(
    _tc_body,
    grid=(GRID_TC,),
    in_specs=[
        pl.BlockSpec((DCHUNK, ENT_TOTAL), lambda i: (i, 0)),
        pl.BlockSpec((DCHUNK, REL_TOTAL), lambda i: (i, 0)),
        pl.BlockSpec((DCHUNK, REL_TOTAL), lambda i: (i, 0)),
    ],
    out_specs=pl.BlockSpec((1, 1), lambda i: (0, 0)),
    out_shape=jax.ShapeDtypeStruct((1, 1), jnp.float32),
    scratch_shapes=[
        pltpu.VMEM((DCHUNK, ENT_TOTAL), jnp.float32),
        pltpu.VMEM((DCHUNK, REL_TOTAL), jnp.float32),
        pltpu.VMEM((DCHUNK, REL_TOTAL), jnp.float32),
    ],
)


def _margin_body(p2_ref, n2_ref, out_ref):
    p2 = jnp.maximum(p2_ref[...], 1e-30)
    n2 = jnp.maximum(n2_ref[...], 1e-30)
    p = p2 * lax.rsqrt(p2)
    n = n2 * lax.rsqrt(n2)
    ml = jnp.sum(jnp.maximum(p - n + MARGIN, 0.0)) / BATCH_SIZE
    out_ref[...] = jnp.reshape(ml, (1, 1))


_tc_margin = pl.pallas_call(
    _margin_body,
    out_shape=jax.ShapeDtypeStruct((1, 1), jnp.float32),
)


def kernel(input, ent_w, rel_w, norm_w):
    hidx = input[:, 0]
    ridx = input[:, 1]
    tidx = input[:, 2]
    dense = _tc_losses(ent_w.T, rel_w.T, norm_w.T)
    score_sq = _build_sc_scores()(hidx, ridx, tidx, ent_w, rel_w, norm_w)
    p2 = score_sq[:BATCH_SIZE].reshape(128, 128)
    n2 = score_sq[BATCH_SIZE:].reshape(128, 128)
    ml = _tc_margin(p2, n2)
    return ml[0, 0] + dense[0, 0]
